# baseline (speedup 1.0000x reference)
import functools

import jax
import jax.numpy as jnp
from jax import lax
from jax.experimental import pallas as pl
from jax.experimental.pallas import tpu as pltpu
from jax.experimental.pallas import tpu_sc as plsc

_P = 4
_L = 4
_K = 2048
_D = 48
_LMBDA = 0.01
_BN = 1024
_N = 36864


def _patchify(x, p):
    B, C, H, W = x.shape
    x = x.reshape(B, C, H // p, p, W // p, p)
    return jnp.transpose(x, (0, 2, 4, 1, 3, 5)).reshape(
        B * (H // p) * (W // p), C * p * p)


def _unpatchify(v, shape, p):
    B, C, H, W = shape
    v = v.reshape(B, H // p, W // p, C, p, p)
    return jnp.transpose(v, (0, 3, 1, 4, 2, 5)).reshape(B, C, H, W)


def _prep_ca(cb):
    c2 = jnp.sum(cb * cb, axis=-1)
    c2h = c2.astype(jnp.bfloat16).astype(jnp.float32)
    c2r = c2 - c2h
    c2m = c2r.astype(jnp.bfloat16).astype(jnp.float32)
    c2l = c2r - c2m
    return jnp.concatenate(
        [-2.0 * cb, c2h[..., None], c2m[..., None], c2l[..., None]], axis=-1)


def _layer_stats(r, ca_ref):
    ones3 = jnp.ones((3, _BN), jnp.float32)
    r_aug = jnp.concatenate([r, ones3], axis=0)
    e = lax.dot_general(ca_ref[...], r_aug, (((1,), (0,)), ((), ())),
                        preferred_element_type=jnp.float32)
    m = jnp.min(e, axis=0)
    mrow = m[None, :]
    ch = 16
    nck = _K // ch
    sacc = jnp.zeros((ch, _BN), jnp.float32)
    iacc = jnp.full((ch, _BN), nck, jnp.int32)
    for ck in range(nck):
        eck = e[ck * ch:(ck + 1) * ch, :]
        iacc = jnp.minimum(iacc, jnp.where(eck == mrow, ck, nck))
        sacc = sacc + jnp.exp(mrow - eck)
    s = jnp.sum(sacc, axis=0)
    idx = jnp.min(iacc * ch
                  + lax.broadcasted_iota(jnp.int32, (ch, _BN), 0), axis=0)
    r2 = jnp.sum(r * r, axis=0)
    vqp = jnp.sum(r2) + jnp.sum(m)
    return idx, jnp.log2(s), vqp


def _acc_scalar(ref, val, i):
    @pl.when(i == 0)
    def _a():
        ref[...] = val[None, None]

    @pl.when(i > 0)
    def _b():
        ref[...] += val[None, None]


def _stage0_body(vt_ref, we_ref, bet_ref, cb_ref,
                 z_ref, idx_ref, rate_ref, vq_ref, ca_ref):
    i = pl.program_id(0)

    @pl.when(i == 0)
    def _init():
        ca_ref[...] = _prep_ca(cb_ref[0])

    z = jnp.tanh(
        lax.dot_general(we_ref[...], vt_ref[...], (((0,), (0,)), ((), ())),
                        preferred_element_type=jnp.float32) + bet_ref[...])
    z_ref[...] = z
    idx, rate, vqp = _layer_stats(z, ca_ref)
    idx_ref[...] = idx
    rate_ref[...] = rate
    _acc_scalar(vq_ref, vqp, i)


def _make_stage_l(l):
    def _body(r_ref, q_ref, ratein_ref, cb_ref,
              rout_ref, idx_ref, rateout_ref, vq_ref, ca_ref):
        i = pl.program_id(0)

        @pl.when(i == 0)
        def _init():
            ca_ref[...] = _prep_ca(cb_ref[0])

        r = r_ref[...] - jnp.transpose(q_ref[...][:, :_D])
        rout_ref[...] = r
        idx, rate, vqp = _layer_stats(r, ca_ref)
        idx_ref[...] = idx
        rateout_ref[...] = ratein_ref[...] + rate
        _acc_scalar(vq_ref, vqp, i)
    return _body


def _final_body(vt_ref, z_ref, r_ref, q_ref, rate_ref, wd_ref, bdt_ref,
                yt_ref, lik_ref, ratesum_ref, mse_ref):
    i = pl.program_id(0)
    r = r_ref[...] - jnp.transpose(q_ref[...][:, :_D])
    yt = lax.dot_general(wd_ref[...], z_ref[...] - r, (((0,), (0,)), ((), ())),
                         preferred_element_type=jnp.float32) + bdt_ref[...]
    yt_ref[...] = yt
    rate = rate_ref[...]
    lik_ref[...] = jnp.exp2(-rate)
    dv = yt - vt_ref[...]
    _acc_scalar(ratesum_ref, jnp.sum(rate), i)
    _acc_scalar(mse_ref, jnp.sum(dv * dv), i)


_NBLK = _N // _BN
_SPEC_T = pl.BlockSpec((_D, _BN), lambda i: (0, i))
_SPEC_R = pl.BlockSpec((_BN, 128), lambda i: (i, 0))
_SPEC_V = pl.BlockSpec((_BN,), lambda i: (i,))
_SPEC_S = pl.BlockSpec((1, 1), lambda i: (0, 0))
_SPEC_CB = pl.BlockSpec((1, _K, _D), lambda i: (0, 0, 0))
_SPEC_W = pl.BlockSpec((_D, _D), lambda i: (0, 0))
_SPEC_B = pl.BlockSpec((_D, 1), lambda i: (0, 0))
_CA_SCRATCH = [pltpu.VMEM((_K, _D + 3), jnp.float32)]

_OUT_T = jax.ShapeDtypeStruct((_D, _N), jnp.float32)
_OUT_I = jax.ShapeDtypeStruct((_N,), jnp.int32)
_OUT_V = jax.ShapeDtypeStruct((_N,), jnp.float32)
_OUT_S = jax.ShapeDtypeStruct((1, 1), jnp.float32)


def _sc_gather(table, idx):
    info = plsc.get_sparse_core_info()
    nw = info.num_cores * info.num_subcores
    bpw = _N // nw
    half = bpw // 2
    mesh = plsc.VectorSubcoreMesh(core_axis_name="c", subcore_axis_name="s")

    @functools.partial(
        pl.kernel, mesh=mesh,
        out_type=jax.ShapeDtypeStruct((_N, 128), jnp.float32),
        scratch_types=[
            pltpu.VMEM((half,), jnp.int32),
            pltpu.VMEM((half, 128), jnp.float32),
            pltpu.SemaphoreType.DMA,
        ],
    )
    def g(table_hbm, idx_hbm, out_hbm, idx_v, rows_v, sem):
        wid = lax.axis_index("s") * info.num_cores + lax.axis_index("c")
        for h in range(2):
            base = wid * bpw + h * half
            pltpu.sync_copy(idx_hbm.at[pl.ds(base, half)], idx_v)
            pltpu.async_copy(table_hbm.at[idx_v], rows_v, sem).wait()
            pltpu.sync_copy(rows_v, out_hbm.at[pl.ds(base, half)])

    return g(table, idx)


@functools.partial(jax.jit, static_argnames=())
def kernel(x, W_enc, b_enc, W_dec, b_dec, codebooks):
    shape = x.shape
    v = _patchify(x, _P)
    vt = v.T
    bet = b_enc.reshape(_D, 1)
    bdt = b_dec.reshape(_D, 1)

    z, idx, rate, vq0 = pl.pallas_call(
        _stage0_body,
        grid=(_NBLK,),
        in_specs=[_SPEC_T, _SPEC_W, _SPEC_B, _SPEC_CB],
        out_specs=[_SPEC_T, _SPEC_V, _SPEC_V, _SPEC_S],
        out_shape=[_OUT_T, _OUT_I, _OUT_V, _OUT_S],
        scratch_shapes=_CA_SCRATCH,
    )(vt, W_enc, bet, codebooks[0:1])

    vqs = [vq0]
    r = z
    cbp = jnp.pad(codebooks, ((0, 0), (0, 0), (0, 128 - _D)))
    for l in range(1, _L):
        q = _sc_gather(cbp[l - 1], idx)
        r, idx, rate, vql = pl.pallas_call(
            _make_stage_l(l),
            grid=(_NBLK,),
            in_specs=[_SPEC_T, _SPEC_R, _SPEC_V, _SPEC_CB],
            out_specs=[_SPEC_T, _SPEC_V, _SPEC_V, _SPEC_S],
            out_shape=[_OUT_T, _OUT_I, _OUT_V, _OUT_S],
            scratch_shapes=_CA_SCRATCH,
        )(r, q, rate, codebooks[l:l + 1])
        vqs.append(vql)

    q = _sc_gather(cbp[_L - 1], idx)
    yt, lik, ratesum, mses = pl.pallas_call(
        _final_body,
        grid=(_NBLK,),
        in_specs=[_SPEC_T, _SPEC_T, _SPEC_T, _SPEC_R, _SPEC_V,
                  _SPEC_W, _SPEC_B],
        out_specs=[_SPEC_T, _SPEC_V, _SPEC_S, _SPEC_S],
        out_shape=[_OUT_T, _OUT_V, _OUT_S, _OUT_S],
    )(vt, z, r, q, rate, W_dec, bdt)

    x_hat = _unpatchify(yt.T, shape, _P)
    n = _N
    rate_mean = ratesum[0, 0] / n
    mse = mses[0, 0] / (n * _D)
    vq_loss = 1.25 * sum(vv[0, 0] for vv in vqs) / (n * _D)
    rd_loss = rate_mean + _LMBDA * mse * (255.0 ** 2)
    loss = rd_loss + vq_loss
    return (x_hat, lik, loss, rd_loss, vq_loss)


# speedup vs baseline: 1.3711x; 1.3711x over previous
import functools

import jax
import jax.numpy as jnp
from jax import lax
from jax.experimental import pallas as pl
from jax.experimental.pallas import tpu as pltpu

_P = 4
_L = 4
_K = 2048
_D = 48
_LMBDA = 0.01
_BN = 2048
_G = 256
_S = _K // _G


def _patchify(x, p):
    B, C, H, W = x.shape
    x = x.reshape(B, C, H // p, p, W // p, p)
    return jnp.transpose(x, (0, 2, 4, 1, 3, 5)).reshape(
        B * (H // p) * (W // p), C * p * p)


def _unpatchify(v, shape, p):
    B, C, H, W = shape
    v = v.reshape(B, H // p, W // p, C, p, p)
    return jnp.transpose(v, (0, 3, 1, 4, 2, 5)).reshape(B, C, H, W)


def _vq_body(vt_ref, we_ref, bet_ref, wd_ref, bdt_ref, cb_ref,
             yt_ref, lik_ref, vq_ref, rate_ref, mse_ref,
             ca_ref, ch_ref, cm_ref, cl_ref):
    i = pl.program_id(0)

    @pl.when(i == 0)
    def _init():
        cb = cb_ref[...]
        c2 = jnp.sum(cb * cb, axis=-1)
        c2h = c2.astype(jnp.bfloat16).astype(jnp.float32)
        c2r = c2 - c2h
        c2m = c2r.astype(jnp.bfloat16).astype(jnp.float32)
        c2l = c2r - c2m
        ca_ref[...] = jnp.concatenate(
            [-2.0 * cb, c2h[..., None], c2m[..., None], c2l[..., None]],
            axis=-1)
        hi = cb.astype(jnp.bfloat16)
        rem = cb - hi.astype(jnp.float32)
        mid = rem.astype(jnp.bfloat16)
        lo = (rem - mid.astype(jnp.float32)).astype(jnp.bfloat16)

        def _pack(p):
            return jnp.concatenate(
                [p[:, sl * _G:(sl + 1) * _G, :] for sl in range(_S)], axis=-1)
        ch_ref[...] = _pack(hi)
        cm_ref[...] = _pack(mid)
        cl_ref[...] = _pack(lo)

    vt = vt_ref[...]
    z = jnp.tanh(
        lax.dot_general(we_ref[...], vt, (((0,), (0,)), ((), ())),
                        preferred_element_type=jnp.float32) + bet_ref[...])
    r = z
    rate = jnp.zeros((_BN,), jnp.float32)
    vq = jnp.float32(0.0)
    ones3 = jnp.ones((3, _BN), jnp.float32)
    for l in range(_L):
        r_aug = jnp.concatenate([r, ones3], axis=0)
        e = lax.dot_general(ca_ref[l], r_aug, (((1,), (0,)), ((), ())),
                            preferred_element_type=jnp.float32)
        m = jnp.min(e, axis=0)
        mrow = m[None, :]
        ch = 16
        nck = _K // ch
        sacc = jnp.zeros((ch, _BN), jnp.float32)
        iacc = jnp.full((ch, _BN), nck, jnp.int32)
        for ck in range(nck):
            eck = e[ck * ch:(ck + 1) * ch, :]
            iacc = jnp.minimum(iacc, jnp.where(eck == mrow, ck, nck))
            sacc = sacc + jnp.exp(mrow - eck)
        s = jnp.sum(sacc, axis=0)
        idx = jnp.min(iacc * ch
                      + lax.broadcasted_iota(jnp.int32, (ch, _BN), 0), axis=0)
        rate = rate + jnp.log2(s)
        r2 = jnp.sum(r * r, axis=0)
        vq = vq + jnp.sum(r2) + jnp.sum(m)
        grp = idx & (_G - 1)
        slot = idx >> 8
        oht = (lax.broadcasted_iota(jnp.int32, (_G, _BN), 0)
               == grp[None, :]).astype(jnp.bfloat16)
        dn = (((0,), (0,)), ((), ()))
        t = (lax.dot_general(cl_ref[l], oht, dn,
                             preferred_element_type=jnp.float32)
             + lax.dot_general(cm_ref[l], oht, dn,
                               preferred_element_type=jnp.float32)
             + lax.dot_general(ch_ref[l], oht, dn,
                               preferred_element_type=jnp.float32))
        b0 = (slot & 1)[None, :] == 1
        b1 = ((slot >> 1) & 1)[None, :] == 1
        b2 = (slot >> 2)[None, :] == 1
        u = [jnp.where(b0, t[(2 * j + 1) * _D:(2 * j + 2) * _D, :],
                       t[2 * j * _D:(2 * j + 1) * _D, :]) for j in range(4)]
        w0 = jnp.where(b1, u[1], u[0])
        w1 = jnp.where(b1, u[3], u[2])
        q = jnp.where(b2, w1, w0)
        r = r - q
    yt = lax.dot_general(wd_ref[...], z - r, (((0,), (0,)), ((), ())),
                         preferred_element_type=jnp.float32) + bdt_ref[...]
    yt_ref[...] = yt
    lik_ref[...] = jnp.exp2(-rate)
    dv = yt - vt
    blk_vq = vq
    blk_rate = jnp.sum(rate)
    blk_mse = jnp.sum(dv * dv)

    @pl.when(i == 0)
    def _store():
        vq_ref[...] = blk_vq[None, None]
        rate_ref[...] = blk_rate[None, None]
        mse_ref[...] = blk_mse[None, None]

    @pl.when(i > 0)
    def _acc():
        vq_ref[...] += blk_vq[None, None]
        rate_ref[...] += blk_rate[None, None]
        mse_ref[...] += blk_mse[None, None]


@functools.partial(jax.jit, static_argnames=())
def kernel(x, W_enc, b_enc, W_dec, b_dec, codebooks):
    shape = x.shape
    v = _patchify(x, _P)
    n = v.shape[0]
    nblk = n // _BN
    vt = v.T

    yt, lik, vqs, rates, mses = pl.pallas_call(
        _vq_body,
        grid=(nblk,),
        in_specs=[
            pl.BlockSpec((_D, _BN), lambda i: (0, i)),
            pl.BlockSpec((_D, _D), lambda i: (0, 0)),
            pl.BlockSpec((_D, 1), lambda i: (0, 0)),
            pl.BlockSpec((_D, _D), lambda i: (0, 0)),
            pl.BlockSpec((_D, 1), lambda i: (0, 0)),
            pl.BlockSpec((_L, _K, _D), lambda i: (0, 0, 0)),
        ],
        out_specs=[
            pl.BlockSpec((_D, _BN), lambda i: (0, i)),
            pl.BlockSpec((_BN,), lambda i: (i,)),
            pl.BlockSpec((1, 1), lambda i: (0, 0)),
            pl.BlockSpec((1, 1), lambda i: (0, 0)),
            pl.BlockSpec((1, 1), lambda i: (0, 0)),
        ],
        out_shape=[
            jax.ShapeDtypeStruct((_D, n), jnp.float32),
            jax.ShapeDtypeStruct((n,), jnp.float32),
            jax.ShapeDtypeStruct((1, 1), jnp.float32),
            jax.ShapeDtypeStruct((1, 1), jnp.float32),
            jax.ShapeDtypeStruct((1, 1), jnp.float32),
        ],
        scratch_shapes=[
            pltpu.VMEM((_L, _K, _D + 3), jnp.float32),
            pltpu.VMEM((_L, _G, _S * _D), jnp.bfloat16),
            pltpu.VMEM((_L, _G, _S * _D), jnp.bfloat16),
            pltpu.VMEM((_L, _G, _S * _D), jnp.bfloat16),
        ],
    )(vt, W_enc, b_enc.reshape(_D, 1), W_dec, b_dec.reshape(_D, 1),
      codebooks)

    x_hat = _unpatchify(yt.T, shape, _P)
    rate = rates[0, 0] / n
    mse = mses[0, 0] / (n * _D)
    vq_loss = 1.25 * vqs[0, 0] / (n * _D)
    rd_loss = rate + _LMBDA * mse * (255.0 ** 2)
    loss = rd_loss + vq_loss
    return (x_hat, lik, loss, rd_loss, vq_loss)


# speedup vs baseline: 1.3747x; 1.0027x over previous
import functools

import jax
import jax.numpy as jnp
from jax import lax
from jax.experimental import pallas as pl
from jax.experimental.pallas import tpu as pltpu

_P = 4
_L = 4
_K = 2048
_D = 48
_LMBDA = 0.01
_BN = 2048
_G = 512
_S = _K // _G


def _patchify(x, p):
    B, C, H, W = x.shape
    x = x.reshape(B, C, H // p, p, W // p, p)
    return jnp.transpose(x, (0, 2, 4, 1, 3, 5)).reshape(
        B * (H // p) * (W // p), C * p * p)


def _unpatchify(v, shape, p):
    B, C, H, W = shape
    v = v.reshape(B, H // p, W // p, C, p, p)
    return jnp.transpose(v, (0, 3, 1, 4, 2, 5)).reshape(B, C, H, W)


def _vq_body(vt_ref, we_ref, bet_ref, wd_ref, bdt_ref, cb_ref,
             yt_ref, lik_ref, vq_ref, rate_ref, mse_ref,
             ca_ref, ch_ref, cm_ref, cl_ref):
    i = pl.program_id(0)

    @pl.when(i == 0)
    def _init():
        cb = cb_ref[...]
        c2 = jnp.sum(cb * cb, axis=-1)
        c2h = c2.astype(jnp.bfloat16).astype(jnp.float32)
        c2r = c2 - c2h
        c2m = c2r.astype(jnp.bfloat16).astype(jnp.float32)
        c2l = c2r - c2m
        ca_ref[...] = jnp.concatenate(
            [-2.0 * cb, c2h[..., None], c2m[..., None], c2l[..., None]],
            axis=-1)
        hi = cb.astype(jnp.bfloat16)
        rem = cb - hi.astype(jnp.float32)
        mid = rem.astype(jnp.bfloat16)
        lo = (rem - mid.astype(jnp.float32)).astype(jnp.bfloat16)

        def _pack(p):
            return jnp.concatenate(
                [p[:, sl * _G:(sl + 1) * _G, :] for sl in range(_S)], axis=-1)
        ch_ref[...] = _pack(hi)
        cm_ref[...] = _pack(mid)
        cl_ref[...] = _pack(lo)

    vt = vt_ref[...]
    z = jnp.tanh(
        lax.dot_general(we_ref[...], vt, (((0,), (0,)), ((), ())),
                        preferred_element_type=jnp.float32) + bet_ref[...])
    r = z
    rate = jnp.zeros((_BN,), jnp.float32)
    vq = jnp.float32(0.0)
    ones3 = jnp.ones((3, _BN), jnp.float32)
    for l in range(_L):
        r_aug = jnp.concatenate([r, ones3], axis=0)
        e = lax.dot_general(ca_ref[l], r_aug, (((1,), (0,)), ((), ())),
                            preferred_element_type=jnp.float32)
        m = jnp.min(e, axis=0)
        mrow = m[None, :]
        ch = 16
        nck = _K // ch
        sacc = jnp.zeros((ch, _BN), jnp.float32)
        iacc = jnp.full((ch, _BN), nck, jnp.int32)
        for ck in range(nck):
            eck = e[ck * ch:(ck + 1) * ch, :]
            iacc = jnp.minimum(iacc, jnp.where(eck == mrow, ck, nck))
            sacc = sacc + jnp.exp(mrow - eck)
        s = jnp.sum(sacc, axis=0)
        idx = jnp.min(iacc * ch
                      + lax.broadcasted_iota(jnp.int32, (ch, _BN), 0), axis=0)
        rate = rate + jnp.log2(s)
        r2 = jnp.sum(r * r, axis=0)
        vq = vq + jnp.sum(r2) + jnp.sum(m)
        grp = idx & (_G - 1)
        slot = idx >> 9
        oht = (lax.broadcasted_iota(jnp.int32, (_G, _BN), 0)
               == grp[None, :]).astype(jnp.bfloat16)
        dn = (((0,), (0,)), ((), ()))
        t = (lax.dot_general(cl_ref[l], oht, dn,
                             preferred_element_type=jnp.float32)
             + lax.dot_general(cm_ref[l], oht, dn,
                               preferred_element_type=jnp.float32)
             + lax.dot_general(ch_ref[l], oht, dn,
                               preferred_element_type=jnp.float32))
        b0 = (slot & 1)[None, :] == 1
        b1 = (slot >> 1)[None, :] == 1
        u = [jnp.where(b0, t[(2 * j + 1) * _D:(2 * j + 2) * _D, :],
                       t[2 * j * _D:(2 * j + 1) * _D, :]) for j in range(2)]
        q = jnp.where(b1, u[1], u[0])
        r = r - q
    yt = lax.dot_general(wd_ref[...], z - r, (((0,), (0,)), ((), ())),
                         preferred_element_type=jnp.float32) + bdt_ref[...]
    yt_ref[...] = yt
    lik_ref[...] = jnp.exp2(-rate)
    dv = yt - vt
    blk_vq = vq
    blk_rate = jnp.sum(rate)
    blk_mse = jnp.sum(dv * dv)

    @pl.when(i == 0)
    def _store():
        vq_ref[...] = blk_vq[None, None]
        rate_ref[...] = blk_rate[None, None]
        mse_ref[...] = blk_mse[None, None]

    @pl.when(i > 0)
    def _acc():
        vq_ref[...] += blk_vq[None, None]
        rate_ref[...] += blk_rate[None, None]
        mse_ref[...] += blk_mse[None, None]


@functools.partial(jax.jit, static_argnames=())
def kernel(x, W_enc, b_enc, W_dec, b_dec, codebooks):
    shape = x.shape
    v = _patchify(x, _P)
    n = v.shape[0]
    nblk = n // _BN
    vt = v.T

    yt, lik, vqs, rates, mses = pl.pallas_call(
        _vq_body,
        grid=(nblk,),
        in_specs=[
            pl.BlockSpec((_D, _BN), lambda i: (0, i)),
            pl.BlockSpec((_D, _D), lambda i: (0, 0)),
            pl.BlockSpec((_D, 1), lambda i: (0, 0)),
            pl.BlockSpec((_D, _D), lambda i: (0, 0)),
            pl.BlockSpec((_D, 1), lambda i: (0, 0)),
            pl.BlockSpec((_L, _K, _D), lambda i: (0, 0, 0)),
        ],
        out_specs=[
            pl.BlockSpec((_D, _BN), lambda i: (0, i)),
            pl.BlockSpec((_BN,), lambda i: (i,)),
            pl.BlockSpec((1, 1), lambda i: (0, 0)),
            pl.BlockSpec((1, 1), lambda i: (0, 0)),
            pl.BlockSpec((1, 1), lambda i: (0, 0)),
        ],
        out_shape=[
            jax.ShapeDtypeStruct((_D, n), jnp.float32),
            jax.ShapeDtypeStruct((n,), jnp.float32),
            jax.ShapeDtypeStruct((1, 1), jnp.float32),
            jax.ShapeDtypeStruct((1, 1), jnp.float32),
            jax.ShapeDtypeStruct((1, 1), jnp.float32),
        ],
        scratch_shapes=[
            pltpu.VMEM((_L, _K, _D + 3), jnp.float32),
            pltpu.VMEM((_L, _G, _S * _D), jnp.bfloat16),
            pltpu.VMEM((_L, _G, _S * _D), jnp.bfloat16),
            pltpu.VMEM((_L, _G, _S * _D), jnp.bfloat16),
        ],
    )(vt, W_enc, b_enc.reshape(_D, 1), W_dec, b_dec.reshape(_D, 1),
      codebooks)

    x_hat = _unpatchify(yt.T, shape, _P)
    rate = rates[0, 0] / n
    mse = mses[0, 0] / (n * _D)
    vq_loss = 1.25 * vqs[0, 0] / (n * _D)
    rd_loss = rate + _LMBDA * mse * (255.0 ** 2)
    loss = rd_loss + vq_loss
    return (x_hat, lik, loss, rd_loss, vq_loss)
